# Initial kernel scaffold; baseline (speedup 1.0000x reference)
import jax
import jax.numpy as jnp
from jax.experimental import pallas as pl


def kernel(x, kv_proj_d, q_proj_d, k_proj_u, q_proj_u, v_proj_u, rope_q, rope_k, o_proj):
    raise NotImplementedError("write your pallas kernel here")


# speedup vs baseline: 8.6578x; 8.6578x over previous
import functools
import math

import jax
import jax.numpy as jnp
from jax import lax
from jax.experimental import pallas as pl
from jax.experimental.pallas import tpu as pltpu


def _dot_nt(a, b):
    return lax.dot_general(a, b, (((1,), (1,)), ((), ())),
                           preferred_element_type=jnp.float32)


def _proj_kernel(x_ref, kvd_ref, qd_ref, ropek_ref, ku_ref, qu_ref, vu_ref,
                 ropeq_ref, kf_ref, vf_ref, qf_ref, qr_ref, kr_ref):
    x = x_ref[...].astype(jnp.bfloat16)
    kv_lat = _dot_nt(x, kvd_ref[...]).astype(jnp.bfloat16)
    q_lat = _dot_nt(x, qd_ref[...]).astype(jnp.bfloat16)
    kr_ref[...] = _dot_nt(x, ropek_ref[...]).astype(jnp.bfloat16)
    kf_ref[...] = _dot_nt(kv_lat, ku_ref[...]).astype(jnp.bfloat16)
    vf_ref[...] = _dot_nt(kv_lat, vu_ref[...]).astype(jnp.bfloat16)
    qf_ref[...] = _dot_nt(q_lat, qu_ref[...]).astype(jnp.bfloat16)
    qr_ref[...] = _dot_nt(q_lat, ropeq_ref[...]).astype(jnp.bfloat16)


def _fused_projections(x2d, weights, *, tm):
    M, hidden = x2d.shape
    kvd, qd, ropek, ku, qu, vu, ropeq = weights
    latent = kvd.shape[0]
    hd_all = ku.shape[0]
    rot_all = ropeq.shape[0]

    full = lambda w: pl.BlockSpec(w.shape, lambda i: (0, 0))
    row_spec = lambda width: pl.BlockSpec((tm, width), lambda i: (i, 0))

    out_shape = (
        jax.ShapeDtypeStruct((M, hd_all), jnp.bfloat16),
        jax.ShapeDtypeStruct((M, hd_all), jnp.bfloat16),
        jax.ShapeDtypeStruct((M, hd_all), jnp.bfloat16),
        jax.ShapeDtypeStruct((M, rot_all), jnp.bfloat16),
        jax.ShapeDtypeStruct((M, rot_all), jnp.bfloat16),
    )
    return pl.pallas_call(
        _proj_kernel,
        out_shape=out_shape,
        grid=(M // tm,),
        in_specs=[row_spec(hidden), full(kvd), full(qd), full(ropek),
                  full(ku), full(qu), full(vu), full(ropeq)],
        out_specs=(row_spec(hd_all), row_spec(hd_all), row_spec(hd_all),
                   row_spec(rot_all), row_spec(rot_all)),
        compiler_params=pltpu.CompilerParams(
            dimension_semantics=("parallel",)),
        cost_estimate=pl.CostEstimate(
            flops=2 * M * hidden * (2 * latent + rot_all)
                  + 2 * M * latent * (3 * hd_all + rot_all),
            transcendentals=0,
            bytes_accessed=4 * M * hidden + 2 * M * (3 * hd_all + 2 * rot_all)),
    )(x2d, kvd, qd, ropek, ku, qu, vu, ropeq)


def _attn_kernel(qf_ref, qr_ref, kf_ref, kr_ref, v_ref, cos_ref, sin_ref,
                 o_ref, *, scale, head_dim, rotary_dim, heads_per_blk):
    T = qf_ref.shape[1]
    cos = cos_ref[...]
    sin = sin_ref[...]
    half = rotary_dim // 2
    hd, rot = head_dim, rotary_dim

    def _rope(x):
        x1 = x[:, :half]
        x2 = x[:, half:]
        return x * cos + jnp.concatenate([-x2, x1], axis=-1) * sin

    q_pos = lax.broadcasted_iota(jnp.int32, (T, T), 0)
    k_pos = lax.broadcasted_iota(jnp.int32, (T, T), 1)
    causal = k_pos <= q_pos

    for h in range(heads_per_blk):
        q_rot = _rope(qr_ref[0, :, h * rot:(h + 1) * rot]
                      .astype(jnp.float32)) * scale
        k_rot = _rope(kr_ref[0, :, h * rot:(h + 1) * rot]
                      .astype(jnp.float32))
        q_tail = qf_ref[0, :, h * hd + rot:(h + 1) * hd]
        q_tail = q_tail.astype(jnp.float32) * scale
        k_tail = kf_ref[0, :, h * hd + rot:(h + 1) * hd].astype(jnp.float32)
        q = jnp.concatenate([q_rot, q_tail], axis=-1).astype(jnp.bfloat16)
        k = jnp.concatenate([k_rot, k_tail], axis=-1).astype(jnp.bfloat16)

        s = _dot_nt(q, k)
        s = jnp.where(causal, s, -jnp.inf)
        m = jnp.max(s, axis=-1, keepdims=True)
        p = jnp.exp(s - m)
        l = jnp.sum(p, axis=-1, keepdims=True)
        acc = jnp.dot(p.astype(jnp.bfloat16), v_ref[0, :, h * hd:(h + 1) * hd],
                      preferred_element_type=jnp.float32)
        o_ref[0, :, h * hd:(h + 1) * hd] = (acc / l).astype(o_ref.dtype)


def _attention(qf, qr, kf, kr, vf, cos, sin, *, B, T, num_heads, head_dim,
               rotary_dim):
    scale = 1.0 / math.sqrt(head_dim)
    hpb = 2
    body = functools.partial(_attn_kernel, scale=scale, head_dim=head_dim,
                             rotary_dim=rotary_dim, heads_per_blk=hpb)

    hd_spec = pl.BlockSpec((1, T, hpb * head_dim), lambda b, p: (b, 0, p))
    rot_spec = pl.BlockSpec((1, T, hpb * rotary_dim), lambda b, p: (b, 0, p))
    cs_spec = pl.BlockSpec((T, rotary_dim), lambda b, p: (0, 0))

    return pl.pallas_call(
        body,
        out_shape=jax.ShapeDtypeStruct((B, T, num_heads * head_dim),
                                       jnp.bfloat16),
        grid=(B, num_heads // hpb),
        in_specs=[hd_spec, rot_spec, hd_spec, rot_spec, hd_spec,
                  cs_spec, cs_spec],
        out_specs=hd_spec,
        compiler_params=pltpu.CompilerParams(
            dimension_semantics=("parallel", "parallel")),
        cost_estimate=pl.CostEstimate(
            flops=4 * B * num_heads * T * T * head_dim,
            transcendentals=B * num_heads * T * T,
            bytes_accessed=2 * B * num_heads * T * (3 * head_dim
                                                    + 2 * rotary_dim)),
    )(qf, qr, kf, kr, vf, cos, sin)


def _oproj_kernel(y_ref, w_ref, o_ref):
    o_ref[...] = _dot_nt(y_ref[...], w_ref[...])


def _out_projection(y2d, w, *, tm, out_dtype):
    M, hidden = y2d.shape
    return pl.pallas_call(
        _oproj_kernel,
        out_shape=jax.ShapeDtypeStruct((M, w.shape[0]), out_dtype),
        grid=(M // tm,),
        in_specs=[pl.BlockSpec((tm, hidden), lambda i: (i, 0)),
                  pl.BlockSpec(w.shape, lambda i: (0, 0))],
        out_specs=pl.BlockSpec((tm, w.shape[0]), lambda i: (i, 0)),
        compiler_params=pltpu.CompilerParams(
            dimension_semantics=("parallel",)),
        cost_estimate=pl.CostEstimate(
            flops=2 * M * hidden * w.shape[0],
            transcendentals=0,
            bytes_accessed=2 * M * hidden * 2 + 4 * M * w.shape[0]),
    )(y2d, w)


def _mla(x, kv_proj_d, q_proj_d, k_proj_u, q_proj_u, v_proj_u, rope_q, rope_k,
         o_proj, *, num_heads, head_dim, rotary_dim, base, scaling_factor):
    B, T, hidden = x.shape
    M = B * T
    x2d = x.reshape(M, hidden)

    bf = lambda w: w.astype(jnp.bfloat16)
    tm = min(256, M)
    kf, vf, qf, qr, kr = _fused_projections(
        x2d, (bf(kv_proj_d), bf(q_proj_d), bf(rope_k), bf(k_proj_u),
              bf(q_proj_u), bf(v_proj_u), bf(rope_q)), tm=tm)

    inv_freq = 1.0 / (base ** (jnp.arange(0, rotary_dim, 2, dtype=jnp.float32)
                               / rotary_dim))
    t_idx = jnp.arange(T, dtype=jnp.float32)
    freqs = jnp.outer(t_idx, inv_freq)
    emb = jnp.concatenate([freqs, freqs], axis=-1)
    cos = jnp.cos(emb) * scaling_factor
    sin = jnp.sin(emb) * scaling_factor

    to_btd = lambda a: a.reshape(B, T, a.shape[-1])
    y = _attention(to_btd(qf), to_btd(qr), to_btd(kf), to_btd(kr), to_btd(vf),
                   cos, sin, B=B, T=T, num_heads=num_heads,
                   head_dim=head_dim, rotary_dim=rotary_dim)

    out = _out_projection(y.reshape(M, hidden), bf(o_proj),
                          tm=min(512, M), out_dtype=x.dtype)
    return out.reshape(B, T, hidden)


def kernel(x, kv_proj_d, q_proj_d, k_proj_u, q_proj_u, v_proj_u, rope_q,
           rope_k, o_proj):
    return _mla(x, kv_proj_d, q_proj_d, k_proj_u, q_proj_u, v_proj_u,
                rope_q, rope_k, o_proj, num_heads=16, head_dim=128,
                rotary_dim=64, base=10000.0, scaling_factor=1.0)


# speedup vs baseline: 9.2905x; 1.0731x over previous
import functools
import math

import jax
import jax.numpy as jnp
from jax import lax
from jax.experimental import pallas as pl
from jax.experimental.pallas import tpu as pltpu


def _dot_nt(a, b):
    return lax.dot_general(a, b, (((1,), (1,)), ((), ())),
                           preferred_element_type=jnp.float32)


def _proj_kernel(x_ref, kvd_ref, qd_ref, ropek_ref, kut_ref, qut_ref, vu_ref,
                 ropeq_ref, kt_ref, vf_ref, qt_ref, qr_ref, kr_ref):
    x = x_ref[...].astype(jnp.bfloat16)
    kv_lat = _dot_nt(x, kvd_ref[...]).astype(jnp.bfloat16)
    q_lat = _dot_nt(x, qd_ref[...]).astype(jnp.bfloat16)
    kr_ref[...] = _dot_nt(x, ropek_ref[...]).astype(jnp.bfloat16)
    kt_ref[...] = _dot_nt(kv_lat, kut_ref[...]).astype(jnp.bfloat16)
    vf_ref[...] = _dot_nt(kv_lat, vu_ref[...]).astype(jnp.bfloat16)
    qt_ref[...] = _dot_nt(q_lat, qut_ref[...]).astype(jnp.bfloat16)
    qr_ref[...] = _dot_nt(q_lat, ropeq_ref[...]).astype(jnp.bfloat16)


def _fused_projections(x2d, weights, *, tm):
    M, hidden = x2d.shape
    kvd, qd, ropek, kut, qut, vu, ropeq = weights
    latent = kvd.shape[0]
    hd_all = vu.shape[0]
    tail_all = kut.shape[0]
    rot_all = ropeq.shape[0]

    full = lambda w: pl.BlockSpec(w.shape, lambda i: (0, 0))
    row_spec = lambda width: pl.BlockSpec((tm, width), lambda i: (i, 0))

    out_shape = (
        jax.ShapeDtypeStruct((M, tail_all), jnp.bfloat16),
        jax.ShapeDtypeStruct((M, hd_all), jnp.bfloat16),
        jax.ShapeDtypeStruct((M, tail_all), jnp.bfloat16),
        jax.ShapeDtypeStruct((M, rot_all), jnp.bfloat16),
        jax.ShapeDtypeStruct((M, rot_all), jnp.bfloat16),
    )
    return pl.pallas_call(
        _proj_kernel,
        out_shape=out_shape,
        grid=(M // tm,),
        in_specs=[row_spec(hidden), full(kvd), full(qd), full(ropek),
                  full(kut), full(qut), full(vu), full(ropeq)],
        out_specs=(row_spec(tail_all), row_spec(hd_all), row_spec(tail_all),
                   row_spec(rot_all), row_spec(rot_all)),
        compiler_params=pltpu.CompilerParams(
            dimension_semantics=("parallel",)),
        cost_estimate=pl.CostEstimate(
            flops=2 * M * hidden * (2 * latent + rot_all)
                  + 2 * M * latent * (2 * tail_all + hd_all + rot_all),
            transcendentals=0,
            bytes_accessed=4 * M * hidden
                           + 2 * M * (2 * tail_all + hd_all + 2 * rot_all)),
    )(x2d, kvd, qd, ropek, kut, qut, vu, ropeq)


def _attn_kernel(qf_ref, qr_ref, kf_ref, kr_ref, v_ref, cos_ref, sin_ref,
                 o_ref, *, scale, head_dim, rotary_dim, heads_per_blk):
    T = qf_ref.shape[1]
    cos = cos_ref[...]
    sin = sin_ref[...]
    half = rotary_dim // 2
    hd, rot = head_dim, rotary_dim

    def _rope(x):
        x1 = x[:, :half]
        x2 = x[:, half:]
        return x * cos + jnp.concatenate([-x2, x1], axis=-1) * sin

    q_pos = lax.broadcasted_iota(jnp.int32, (T, T), 0)
    k_pos = lax.broadcasted_iota(jnp.int32, (T, T), 1)
    causal = k_pos <= q_pos

    tail = hd - rot
    for h in range(heads_per_blk):
        q_rot = _rope(qr_ref[0, :, h * rot:(h + 1) * rot]
                      .astype(jnp.float32)) * scale
        k_rot = _rope(kr_ref[0, :, h * rot:(h + 1) * rot]
                      .astype(jnp.float32))
        q_tail = qf_ref[0, :, h * tail:(h + 1) * tail]
        q_tail = q_tail.astype(jnp.float32) * scale
        k_tail = kf_ref[0, :, h * tail:(h + 1) * tail].astype(jnp.float32)
        q = jnp.concatenate([q_rot, q_tail], axis=-1).astype(jnp.bfloat16)
        k = jnp.concatenate([k_rot, k_tail], axis=-1).astype(jnp.bfloat16)

        s = _dot_nt(q, k)
        s = jnp.where(causal, s, -jnp.inf)
        m = jnp.max(s, axis=-1, keepdims=True)
        p = jnp.exp(s - m)
        l = jnp.sum(p, axis=-1, keepdims=True)
        acc = jnp.dot(p.astype(jnp.bfloat16), v_ref[0, :, h * hd:(h + 1) * hd],
                      preferred_element_type=jnp.float32)
        o_ref[0, :, h * hd:(h + 1) * hd] = (acc / l).astype(o_ref.dtype)


def _attention(qf, qr, kf, kr, vf, cos, sin, *, B, T, num_heads, head_dim,
               rotary_dim):
    scale = 1.0 / math.sqrt(head_dim)
    hpb = 2
    body = functools.partial(_attn_kernel, scale=scale, head_dim=head_dim,
                             rotary_dim=rotary_dim, heads_per_blk=hpb)

    hd_spec = pl.BlockSpec((1, T, hpb * head_dim), lambda b, p: (b, 0, p))
    tail_spec = pl.BlockSpec((1, T, hpb * (head_dim - rotary_dim)),
                             lambda b, p: (b, 0, p))
    rot_spec = pl.BlockSpec((1, T, hpb * rotary_dim), lambda b, p: (b, 0, p))
    cs_spec = pl.BlockSpec((T, rotary_dim), lambda b, p: (0, 0))

    return pl.pallas_call(
        body,
        out_shape=jax.ShapeDtypeStruct((B, T, num_heads * head_dim),
                                       jnp.bfloat16),
        grid=(B, num_heads // hpb),
        in_specs=[tail_spec, rot_spec, tail_spec, rot_spec, hd_spec,
                  cs_spec, cs_spec],
        out_specs=hd_spec,
        compiler_params=pltpu.CompilerParams(
            dimension_semantics=("parallel", "parallel")),
        cost_estimate=pl.CostEstimate(
            flops=4 * B * num_heads * T * T * head_dim,
            transcendentals=B * num_heads * T * T,
            bytes_accessed=2 * B * num_heads * T * (3 * head_dim
                                                    + 2 * rotary_dim)),
    )(qf, qr, kf, kr, vf, cos, sin)


def _oproj_kernel(y_ref, w_ref, o_ref):
    o_ref[...] = _dot_nt(y_ref[...], w_ref[...])


def _out_projection(y2d, w, *, tm, out_dtype):
    M, hidden = y2d.shape
    return pl.pallas_call(
        _oproj_kernel,
        out_shape=jax.ShapeDtypeStruct((M, w.shape[0]), out_dtype),
        grid=(M // tm,),
        in_specs=[pl.BlockSpec((tm, hidden), lambda i: (i, 0)),
                  pl.BlockSpec(w.shape, lambda i: (0, 0))],
        out_specs=pl.BlockSpec((tm, w.shape[0]), lambda i: (i, 0)),
        compiler_params=pltpu.CompilerParams(
            dimension_semantics=("parallel",)),
        cost_estimate=pl.CostEstimate(
            flops=2 * M * hidden * w.shape[0],
            transcendentals=0,
            bytes_accessed=2 * M * hidden * 2 + 4 * M * w.shape[0]),
    )(y2d, w)


def _mla(x, kv_proj_d, q_proj_d, k_proj_u, q_proj_u, v_proj_u, rope_q, rope_k,
         o_proj, *, num_heads, head_dim, rotary_dim, base, scaling_factor):
    B, T, hidden = x.shape
    M = B * T
    x2d = x.reshape(M, hidden)

    bf = lambda w: w.astype(jnp.bfloat16)
    latent = kv_proj_d.shape[0]
    tails = lambda w: w.reshape(num_heads, head_dim, latent)[:, rotary_dim:, :
                               ].reshape(-1, latent)
    tm = min(256, M)
    kf, vf, qf, qr, kr = _fused_projections(
        x2d, (bf(kv_proj_d), bf(q_proj_d), bf(rope_k), bf(tails(k_proj_u)),
              bf(tails(q_proj_u)), bf(v_proj_u), bf(rope_q)), tm=tm)

    inv_freq = 1.0 / (base ** (jnp.arange(0, rotary_dim, 2, dtype=jnp.float32)
                               / rotary_dim))
    t_idx = jnp.arange(T, dtype=jnp.float32)
    freqs = jnp.outer(t_idx, inv_freq)
    emb = jnp.concatenate([freqs, freqs], axis=-1)
    cos = jnp.cos(emb) * scaling_factor
    sin = jnp.sin(emb) * scaling_factor

    to_btd = lambda a: a.reshape(B, T, a.shape[-1])
    y = _attention(to_btd(qf), to_btd(qr), to_btd(kf), to_btd(kr), to_btd(vf),
                   cos, sin, B=B, T=T, num_heads=num_heads,
                   head_dim=head_dim, rotary_dim=rotary_dim)

    out = _out_projection(y.reshape(M, hidden), bf(o_proj),
                          tm=min(512, M), out_dtype=x.dtype)
    return out.reshape(B, T, hidden)


def kernel(x, kv_proj_d, q_proj_d, k_proj_u, q_proj_u, v_proj_u, rope_q,
           rope_k, o_proj):
    return _mla(x, kv_proj_d, q_proj_d, k_proj_u, q_proj_u, v_proj_u,
                rope_q, rope_k, o_proj, num_heads=16, head_dim=128,
                rotary_dim=64, base=10000.0, scaling_factor=1.0)
